# Initial kernel scaffold; baseline (speedup 1.0000x reference)
import jax
import jax.numpy as jnp
from jax.experimental import pallas as pl


def kernel(x, W, b):
    raise NotImplementedError("write your pallas kernel here")


# speedup vs baseline: 4.0943x; 4.0943x over previous
import jax
import jax.numpy as jnp
from jax import lax
from jax.experimental import pallas as pl

B, C, H, W_SP, E, K = 4, 64, 128, 128, 64, 8
S = H * W_SP
TBLK = 2048
NS = S // TBLK


def _router_body(x_ref, w_ref, b_ref, logits_ref, weights_ref, idx_ref, mask_ref):
    xb = x_ref[0]
    l = jnp.dot(w_ref[...], xb, preferred_element_type=jnp.float32)
    l = l + b_ref[...]
    logits_ref[0] = l

    m = jnp.max(l, axis=0, keepdims=True)
    e = jnp.exp(l - m)
    z = jnp.sum(e, axis=0, keepdims=True)
    p = e / z

    ii = lax.broadcasted_iota(jnp.int32, (C, TBLK), 0)
    vals, idxs = [], []
    cur = p
    for _ in range(K):
        mk = jnp.max(cur, axis=0, keepdims=True)
        sel = cur == mk
        ik = jnp.min(jnp.where(sel, ii, C), axis=0, keepdims=True)
        vals.append(mk)
        idxs.append(ik)
        cur = jnp.where(ii == ik, -1.0, cur)

    wv = jnp.concatenate(vals, axis=0)
    iv = jnp.concatenate(idxs, axis=0)
    weights_ref[0] = wv / jnp.sum(wv, axis=0, keepdims=True)
    idx_ref[0] = iv

    ee = lax.broadcasted_iota(jnp.int32, (E, K, TBLK), 0)
    mask_ref[...] = (iv[None] == ee).astype(jnp.int32)


def kernel(x, W, b):
    xr = x.reshape(B, C, S)
    br = b.reshape(C, 1)
    grid = (B, NS)
    logits, weights, idx, mask = pl.pallas_call(
        _router_body,
        grid=grid,
        in_specs=[
            pl.BlockSpec((1, C, TBLK), lambda bb, s: (bb, 0, s)),
            pl.BlockSpec((C, C), lambda bb, s: (0, 0)),
            pl.BlockSpec((C, 1), lambda bb, s: (0, 0)),
        ],
        out_specs=[
            pl.BlockSpec((1, C, TBLK), lambda bb, s: (bb, 0, s)),
            pl.BlockSpec((1, K, TBLK), lambda bb, s: (bb, 0, s)),
            pl.BlockSpec((1, K, TBLK), lambda bb, s: (bb, 0, s)),
            pl.BlockSpec((E, K, TBLK), lambda bb, s: (0, 0, bb * NS + s)),
        ],
        out_shape=[
            jax.ShapeDtypeStruct((B, C, S), jnp.float32),
            jax.ShapeDtypeStruct((B, K, S), jnp.float32),
            jax.ShapeDtypeStruct((B, K, S), jnp.int32),
            jax.ShapeDtypeStruct((E, K, B * S), jnp.int32),
        ],
    )(xr, W, br)
    return (
        logits.reshape(B, C, H, W_SP),
        weights.reshape(B, K, H, W_SP),
        idx.reshape(B, K, H, W_SP),
        mask,
    )
